# Initial kernel scaffold; baseline (speedup 1.0000x reference)
import jax
import jax.numpy as jnp
from jax.experimental import pallas as pl


def kernel(fea, pred, contrast_loss_input, pesudo_label, queues):
    raise NotImplementedError("write your pallas kernel here")


# speedup vs baseline: 2.5334x; 2.5334x over previous
import functools

import jax
import jax.numpy as jnp
from jax.experimental import pallas as pl
from jax.experimental.pallas import tpu as pltpu

_TEMP = 0.2
_QB = 2048


def _seg_kernel(fea_ref, pred_ref, sums_ref, cnt_ref, *, num_classes, hw):
    b = pl.program_id(0)
    p = pred_ref[0]
    best_v = p[0:1, :]
    best_i = jnp.zeros_like(best_v)
    for c in range(1, num_classes):
        v = p[c : c + 1, :]
        take = v > best_v
        best_v = jnp.where(take, v, best_v)
        best_i = jnp.where(take, jnp.float32(c), best_i)
    iota8 = jax.lax.broadcasted_iota(jnp.int32, (8, hw), 0).astype(jnp.float32)
    onehot = (iota8 == best_i).astype(jnp.float32)
    f = fea_ref[0]
    part = jax.lax.dot_general(
        f, onehot, (((1,), (1,)), ((), ())), preferred_element_type=jnp.float32
    )
    ones = jnp.ones((1, hw), jnp.float32)
    cnt = jax.lax.dot_general(
        ones, onehot, (((1,), (1,)), ((), ())), preferred_element_type=jnp.float32
    )

    @pl.when(b == 0)
    def _():
        sums_ref[...] = part
        cnt_ref[...] = cnt

    @pl.when(b > 0)
    def _():
        sums_ref[...] += part
        cnt_ref[...] += cnt


def _contrast_kernel(q_ref, k_ref, pos_ref, neg_ref, *, num_classes, queue_len):
    j = pl.program_id(0)

    @pl.when(j == 0)
    def _():
        pos_ref[...] = jnp.zeros_like(pos_ref)
        neg_ref[...] = jnp.zeros_like(neg_ref)

    blk = q_ref[...]
    s = jnp.sum(blk, axis=0)
    lane = jax.lax.broadcasted_iota(jnp.int32, blk.shape[1:], 1)
    valid = (j * _QB + lane) < queue_len
    for c in range(num_classes):
        x = blk[c]
        k = k_ref[:, c : c + 1]
        e_pos = jnp.where(valid, jnp.exp(x * k), 0.0)
        e_neg = jnp.where(valid, jnp.exp((s - x) * k), 0.0)
        pos_ref[:, c : c + 1] += jnp.sum(e_pos, axis=1, keepdims=True)
        neg_ref[:, c : c + 1] += jnp.sum(e_neg, axis=1, keepdims=True)


def _mask_kernel(
    plab_ref, pos_ref, neg_ref, k_ref, q0_ref, cnt_ref, th_ref, out_ref,
    *, num_classes, in_planes
):
    lse = jnp.log(pos_ref[...] + neg_ref[...])
    l0 = k_ref[...] * q0_ref[...]
    pmap = plab_ref[...]
    best_v = pmap[:, 0, :]
    best_i = jnp.zeros_like(best_v)
    for c in range(1, num_classes):
        v = pmap[:, c, :]
        take = v > best_v
        best_v = jnp.where(take, v, best_v)
        best_i = jnp.where(take, jnp.float32(c), best_i)
    out = best_i
    for c in range(num_classes):
        loss_c = (jnp.sum(lse[:, c]) - jnp.sum(l0[:, c])) / jnp.float32(in_planes)
        drop = jnp.logical_or(cnt_ref[c] <= 0.0, loss_c > th_ref[c])
        out = jnp.where(
            jnp.logical_and(drop, best_i == jnp.float32(c)), jnp.float32(-1.0), out
        )
    out_ref[...] = out


@jax.jit
def kernel(fea, pred, contrast_loss_input, pesudo_label, queues):
    bsz, in_planes, hgt, wid = fea.shape
    num_classes = pred.shape[1]
    queue_len = queues.shape[2]
    hw = hgt * wid

    fea3 = fea.reshape(bsz, in_planes, hw)
    pred3 = pred.reshape(bsz, num_classes, hw)
    plab3 = pesudo_label.reshape(bsz, num_classes, hw)

    sums8, cnt8 = pl.pallas_call(
        functools.partial(_seg_kernel, num_classes=num_classes, hw=hw),
        grid=(bsz,),
        in_specs=[
            pl.BlockSpec((1, in_planes, hw), lambda b: (b, 0, 0)),
            pl.BlockSpec((1, num_classes, hw), lambda b: (b, 0, 0)),
        ],
        out_specs=[
            pl.BlockSpec((in_planes, 8), lambda b: (0, 0)),
            pl.BlockSpec((1, 8), lambda b: (0, 0)),
        ],
        out_shape=[
            jax.ShapeDtypeStruct((in_planes, 8), jnp.float32),
            jax.ShapeDtypeStruct((1, 8), jnp.float32),
        ],
    )(fea3, pred3)

    sums = sums8[:, :num_classes]
    cnt = cnt8[0, :num_classes]
    means = sums / jnp.where(cnt > 0, cnt, 1.0)[None, :]
    norm = jnp.sqrt(jnp.sum(means * means, axis=0, keepdims=True))
    keys_t = means / jnp.maximum(norm, 1e-12)
    keys_scaled = keys_t * jnp.float32(1.0 / _TEMP)

    nq = pl.cdiv(queue_len, _QB)
    pos_t, neg_t = pl.pallas_call(
        functools.partial(
            _contrast_kernel, num_classes=num_classes, queue_len=queue_len
        ),
        grid=(nq,),
        in_specs=[
            pl.BlockSpec((num_classes, in_planes, _QB), lambda j: (0, 0, j)),
            pl.BlockSpec((in_planes, num_classes), lambda j: (0, 0)),
        ],
        out_specs=[
            pl.BlockSpec((in_planes, num_classes), lambda j: (0, 0)),
            pl.BlockSpec((in_planes, num_classes), lambda j: (0, 0)),
        ],
        out_shape=[
            jax.ShapeDtypeStruct((in_planes, num_classes), jnp.float32),
            jax.ShapeDtypeStruct((in_planes, num_classes), jnp.float32),
        ],
    )(queues, keys_scaled)

    q0_t = queues[:, :, 0].T

    out = pl.pallas_call(
        functools.partial(
            _mask_kernel, num_classes=num_classes, in_planes=in_planes
        ),
        grid=(1,),
        in_specs=[
            pl.BlockSpec((bsz, num_classes, hw), lambda i: (0, 0, 0)),
            pl.BlockSpec((in_planes, num_classes), lambda i: (0, 0)),
            pl.BlockSpec((in_planes, num_classes), lambda i: (0, 0)),
            pl.BlockSpec((in_planes, num_classes), lambda i: (0, 0)),
            pl.BlockSpec((in_planes, num_classes), lambda i: (0, 0)),
            pl.BlockSpec(memory_space=pltpu.SMEM),
            pl.BlockSpec(memory_space=pltpu.SMEM),
        ],
        out_specs=pl.BlockSpec((bsz, hw), lambda i: (0, 0)),
        out_shape=jax.ShapeDtypeStruct((bsz, hw), jnp.float32),
    )(plab3, pos_t, neg_t, keys_scaled, q0_t, cnt, contrast_loss_input)

    return out.reshape(bsz, hgt, wid)
